# baseline (speedup 1.0000x reference)
import functools

import jax
import jax.numpy as jnp
from jax import lax
from jax.experimental import pallas as pl
from jax.experimental.pallas import tpu as pltpu
from jax.experimental.pallas import tpu_sc as plsc

_SCALE = 10.0
_DIM = 64
_NBATCH = 16384
_SEQ = 50
_NC = 2
_NS = 16
_NW = _NC * _NS
_BPW = _NBATCH // _NW
_BC = 128
_CBW = _BPW // _BC
_UNITS = _SEQ * _CBW
_NB = 4
_LOOKAHEAD = 2
_DT = _DIM // 8
_TROW = 129


def _sc_body(w_hbm, x_hbm, out_hbm, idx_v, idx_t, *bufs_sems):
    gbufs = bufs_sems[:_NB]
    tbufs = bufs_sems[_NB:2 * _NB]
    gs = bufs_sems[2 * _NB:3 * _NB]
    os_ = bufs_sems[3 * _NB:]
    wid = lax.axis_index("s") * _NC + lax.axis_index("c")
    b_base = wid * _BPW
    bc_base = wid * _CBW

    pltpu.sync_copy(x_hbm.at[pl.ds(b_base * _SEQ, _BPW * _SEQ)], idx_v)
    iota = lax.iota(jnp.int32, 16)
    zero16 = jnp.zeros((16,), jnp.int32)
    fbase = [
        (c8 * 2 + lax.shift_right_logical(iota, 3)) * (8 * _TROW)
        + lax.bitwise_and(iota, 7) * _TROW
        for c8 in range(_DIM // 16)
    ]

    def idx_t_row(s, carry):
        for c in range(_BPW // 16):
            src = plsc.load_gather(idx_v, [(c * 16 + iota) * _SEQ + s])
            idx_t[s, pl.ds(c * 16, 16)] = src
        return carry

    lax.fori_loop(0, _SEQ, idx_t_row, 0)

    def gather(s, cb, b):
        return pltpu.make_async_copy(
            w_hbm.at[idx_t.at[s, pl.ds(cb * _BC, _BC)]],
            gbufs[b],
            gs[b],
        )

    def out_copies(s, cb, b):
        return [
            pltpu.make_async_copy(
                tbufs[b].at[dt, pl.ds(0, 8), pl.ds(0, _BC)],
                out_hbm.at[s, dt, bc_base + cb],
                os_[b],
            )
            for dt in range(_DT)
        ]

    for u in range(_LOOKAHEAD):
        gather(0, u, u).start()

    def unit(g, i):
        s = g
        cb = i
        b = i
        b2 = (i + _LOOKAHEAD) % _NB
        s2 = g + (i + _LOOKAHEAD) // _NB
        cb2 = (i + _LOOKAHEAD) % _NB
        sp = g + (i - _LOOKAHEAD) // _NB
        cbp = (i - _LOOKAHEAD) % _NB

        gather(s, cb, b).wait()

        def fire_next():
            @pl.when((g >= 1) if i < _LOOKAHEAD else (g >= 0))
            def _():
                for c in out_copies(sp, cbp, b2):
                    c.wait()

            gather(s2, cb2, b2).start()

        if _LOOKAHEAD <= i:
            @pl.when(s2 < _SEQ)
            def _():
                fire_next()
        else:
            fire_next()

        def brow(r, carry):
            roff = fbase[0] + r
            v = gbufs[b][r, pl.ds(0, 16)] * _SCALE
            plsc.store_scatter(tbufs[b], [zero16, zero16, roff], v)
            for c8 in range(1, _DIM // 16):
                roff = fbase[c8] + r
                v = gbufs[b][r, pl.ds(c8 * 16, 16)] * _SCALE
                plsc.store_scatter(tbufs[b], [zero16, zero16, roff], v)
            return carry

        lax.fori_loop(0, _BC, brow, 0, unroll=4)

        for cpy in out_copies(s, cb, b):
            cpy.start()

    def outer(g, carry):
        for i in range(_NB):
            unit(g, i)
        return carry

    lax.fori_loop(0, _UNITS // _NB, outer, 0)

    for i in range(_NB):
        for cpy in out_copies(_SEQ - 1, i, i):
            cpy.wait()


@functools.partial(jax.jit, static_argnames=())
def kernel(x, weight):
    idx = x.reshape(-1).astype(jnp.int32)
    mesh = plsc.VectorSubcoreMesh(core_axis_name="c", subcore_axis_name="s")
    arr5 = pl.kernel(
        _sc_body,
        mesh=mesh,
        out_type=jax.ShapeDtypeStruct((_SEQ, _DT, _NBATCH // _BC, 8, _BC),
                                      jnp.float32),
        scratch_types=[
            pltpu.VMEM((_BPW * _SEQ,), jnp.int32),
            pltpu.VMEM((_SEQ, _BPW), jnp.int32),
        ]
        + [pltpu.VMEM((_BC, _DIM), jnp.float32) for _ in range(_NB)]
        + [pltpu.VMEM((_DT, 8, _TROW), jnp.float32) for _ in range(_NB)]
        + [pltpu.SemaphoreType.DMA for _ in range(2 * _NB)],
        compiler_params=pltpu.CompilerParams(
            use_tc_tiling_on_sc=False, needs_layout_passes=False
        ),
    )(weight, idx)
    a7 = jnp.transpose(arr5, (2, 4, 0, 1, 3))
    return a7.reshape(_NBATCH, _SEQ, _DIM)


# speedup vs baseline: 1.4703x; 1.4703x over previous
import functools

import jax
import jax.numpy as jnp
from jax import lax
from jax.experimental import pallas as pl
from jax.experimental.pallas import tpu as pltpu
from jax.experimental.pallas import tpu_sc as plsc

_SCALE = 10.0
_DIM = 64
_NBATCH = 16384
_SEQ = 50
_NC = 2
_NS = 16
_NW = _NC * _NS
_BATCH_PER_W = _NBATCH // _NW
_SUPB = 4
_SUP = _SUPB * _SEQ
_SUPERS = _BATCH_PER_W // _SUPB
_NB = 8
_LOOKAHEAD = 4
_SPLITS = ((0, 128), (128, _SUP - 128))


def _sc_body(w_hbm, x_hbm, out3_hbm, idx_v, *bufs_sems):
    bufs = bufs_sems[:_NB]
    gs = bufs_sems[_NB:2 * _NB]
    os_ = bufs_sems[2 * _NB:]
    wid = lax.axis_index("s") * _NC + lax.axis_index("c")
    b_base = wid * _BATCH_PER_W

    pltpu.sync_copy(
        x_hbm.at[pl.ds(b_base * _SEQ, _BATCH_PER_W * _SEQ)], idx_v
    )

    def gather(s, b, q):
        off, cnt = _SPLITS[q]
        return pltpu.make_async_copy(
            w_hbm.at[idx_v.at[pl.ds(s * _SUP + off, cnt)]],
            bufs[b].at[pl.ds(off, cnt)],
            gs[b],
        )

    def out_copies(s, b):
        return [
            pltpu.make_async_copy(
                bufs[b].at[pl.ds(k * _SEQ, _SEQ)],
                out3_hbm.at[
                    b_base + s * _SUPB + k, pl.ds(0, _SEQ), pl.ds(0, _DIM)
                ],
                os_[b],
            )
            for k in range(_SUPB)
        ]

    def fire(s, b):
        for q in range(len(_SPLITS)):
            gather(s, b, q).start()

    for b in range(_LOOKAHEAD):
        fire(b, b)

    def sup_iter(g, i):
        s = g * _NB + i
        b = i
        b2 = (i + _LOOKAHEAD) % _NB
        for q in range(len(_SPLITS)):
            gather(s, b, q).wait()

        @pl.when(s + _LOOKAHEAD < _SUPERS)
        def _():
            @pl.when(s >= _LOOKAHEAD)
            def _():
                for c in out_copies(s - _NB + _LOOKAHEAD, b2):
                    c.wait()

            fire(s + _LOOKAHEAD, b2)

        def scale_row(r, c2):
            for c in range(_DIM // 16):
                sl = pl.ds(c * 16, 16)
                bufs[b][r, sl] = bufs[b][r, sl] * _SCALE
            return c2

        lax.fori_loop(0, _SUP, scale_row, 0, unroll=8)

        for c in out_copies(s, b):
            c.start()

    def outer(g, carry):
        for i in range(_NB):
            sup_iter(g, i)
        return carry

    lax.fori_loop(0, _SUPERS // _NB, outer, 0)

    for i in range(_NB):
        for c in out_copies(_SUPERS - _NB + i, i):
            c.wait()


@functools.partial(jax.jit, static_argnames=())
def kernel(x, weight):
    idx = x.reshape(-1).astype(jnp.int32)
    mesh = plsc.VectorSubcoreMesh(core_axis_name="c", subcore_axis_name="s")
    padded = pl.kernel(
        _sc_body,
        mesh=mesh,
        out_type=jax.ShapeDtypeStruct((_NBATCH, 56, 128), jnp.float32),
        scratch_types=[
            pltpu.VMEM((_BATCH_PER_W * _SEQ,), jnp.int32),
        ]
        + [pltpu.VMEM((_SUP, _DIM), jnp.float32) for _ in range(_NB)]
        + [pltpu.SemaphoreType.DMA for _ in range(2 * _NB)],
        compiler_params=pltpu.CompilerParams(use_tc_tiling_on_sc=False),
    )(weight, idx)
    return padded[:, :_SEQ, :_DIM]
